# baseline (speedup 1.0000x reference)
import functools

import jax
import jax.numpy as jnp
from jax import lax
from jax.experimental import pallas as pl
from jax.experimental.pallas import tpu as pltpu
from jax.experimental.pallas import tpu_sc as plsc

B, D1, D2 = 8, 8192, 576
NC, NS, L = 2, 16, 16
RSPLIT = 4
RPT = D1 // RSPLIT
NG = D2 // L
RCHUNK = 64
D2P = 640
NCHUNK = RPT // RCHUNK
U = 8

_mesh = plsc.VectorSubcoreMesh(
    core_axis_name="c", subcore_axis_name="s", num_cores=NC, num_subcores=NS
)


_scratch_types = [
    pltpu.VMEM((RCHUNK, D2), jnp.float32),
    pltpu.VMEM((RCHUNK, D2), jnp.float32),
    pltpu.VMEM((D2P,), jnp.float32),
    pltpu.VMEM((D2P,), jnp.int32),
    pltpu.VMEM((D2P,), jnp.float32),
    pltpu.VMEM((D2P,), jnp.int32),
    pltpu.VMEM((D2,), jnp.int32),
    pltpu.VMEM_SHARED((NS, 1, D2P), jnp.float32),
    pltpu.VMEM_SHARED((NS, 1, D2P), jnp.int32),
    pltpu.SemaphoreType.DMA,
    pltpu.SemaphoreType.DMA,
]


def _argmin_body(x_hbm, out_hbm, buf0, buf1, stv, sti, mgv, mgi, outv,
                 shv, shi, sem0, sem1):
    cid = lax.axis_index("c")
    sid = lax.axis_index("s")
    batch = cid * (NS // RSPLIT) + sid // RSPLIT
    rbase = (sid % RSPLIT) * RPT

    bufs = (buf0, buf1)
    sems = (sem0, sem1)

    def start_dma(i, buf, sem):
        return pltpu.async_copy(
            x_hbm.at[batch, pl.ds(rbase + i * RCHUNK, RCHUNK), :], buf, sem
        )

    def init_body(g, _):
        goff = g * L
        stv[pl.ds(goff, L)] = jnp.full((L,), jnp.inf, jnp.float32)
        sti[pl.ds(goff, L)] = jnp.zeros((L,), jnp.int32)
        return 0

    lax.fori_loop(0, NG, init_body, 0)

    def wait_dma(buf, sem):
        pltpu.make_async_copy(
            x_hbm.at[batch, pl.ds(0, RCHUNK), :], buf, sem
        ).wait()

    def process(buf, ci):
        rv = jnp.full((L,), rbase + ci * RCHUNK, jnp.int32)
        rvs = [rv + jnp.full((L,), sb * U, jnp.int32) for sb in range(RCHUNK // U)]

        def g_body(g, _):
            goff = g * L
            mv = stv[pl.ds(goff, L)]
            mi = sti[pl.ds(goff, L)]
            for sb in range(RCHUNK // U):
                v = [buf[sb * U + j, pl.ds(goff, L)] for j in range(U)]
                ix = [jnp.full((L,), j, jnp.int32) for j in range(U)]
                n = U
                while n > 1:
                    nv, ni = [], []
                    for p in range(0, n, 2):
                        m = v[p] <= v[p + 1]
                        nv.append(jnp.minimum(v[p], v[p + 1]))
                        ni.append(jnp.where(m, ix[p], ix[p + 1]))
                    v, ix, n = nv, ni, n // 2
                m = v[0] < mv
                mv = jnp.where(m, v[0], mv)
                mi = jnp.where(m, rvs[sb] + ix[0], mi)
            stv[pl.ds(goff, L)] = mv
            sti[pl.ds(goff, L)] = mi
            return 0

        lax.fori_loop(0, NG, g_body, 0)

    start_dma(0, bufs[0], sems[0])

    def outer_body(t, _):
        c0 = pl.multiple_of(t * 2, 2)
        wait_dma(bufs[0], sems[0])
        start_dma(c0 + 1, bufs[1], sems[1])
        process(bufs[0], c0)

        @pl.when(t < NCHUNK // 2 - 1)
        def _prefetch():
            start_dma(c0 + 2, bufs[0], sems[0])

        wait_dma(bufs[1], sems[1])
        process(bufs[1], c0 + 1)
        return 0

    lax.fori_loop(0, NCHUNK // 2, outer_body, 0)

    pltpu.sync_copy(stv, shv.at[sid, 0, :])
    pltpu.sync_copy(sti, shi.at[sid, 0, :])
    plsc.subcore_barrier()

    @pl.when(sid % RSPLIT == 0)
    def _merge():
        for j in range(1, RSPLIT):
            pltpu.sync_copy(shv.at[sid + j, 0, :], mgv)
            pltpu.sync_copy(shi.at[sid + j, 0, :], mgi)

            def m_body(g, _):
                goff = g * L
                mv = stv[pl.ds(goff, L)]
                mi = sti[pl.ds(goff, L)]
                ov = mgv[pl.ds(goff, L)]
                oi = mgi[pl.ds(goff, L)]
                m = ov < mv
                stv[pl.ds(goff, L)] = jnp.where(m, ov, mv)
                sti[pl.ds(goff, L)] = jnp.where(m, oi, mi)
                return 0

            lax.fori_loop(0, NG, m_body, 0)

        def o_body(g, _):
            goff = g * L
            outv[pl.ds(goff, L)] = sti[pl.ds(goff, L)]
            return 0

        lax.fori_loop(0, NG, o_body, 0)
        pltpu.sync_copy(outv, out_hbm.at[batch, 0, :])


_argmin_kernel = pl.kernel(
    _argmin_body,
    out_type=jax.ShapeDtypeStruct((B, 1, D2), jnp.int32),
    mesh=_mesh,
    scratch_types=_scratch_types,
)


_TCK = 512
_TCNB = D1 // _TCK


def _tc_body(x_ref, o_ref, mv_ref, mi_ref):
    k = pl.program_id(1)
    xb = x_ref[0]
    bmin = jnp.min(xb, axis=0, keepdims=True)
    iota = lax.broadcasted_iota(jnp.int32, (_TCK, D2), 0)
    bidx = jnp.min(jnp.where(xb == bmin, iota, D1), axis=0, keepdims=True)

    @pl.when(k == 0)
    def _init():
        mv_ref[...] = bmin
        mi_ref[...] = bidx

    @pl.when(k > 0)
    def _merge():
        cur = mv_ref[...]
        m = bmin < cur
        mv_ref[...] = jnp.where(m, bmin, cur)
        mi_ref[...] = jnp.where(m, bidx + k * _TCK, mi_ref[...])

    @pl.when(k == _TCNB - 1)
    def _out():
        o_ref[0] = mi_ref[...]


def _tc_argmin(x):
    nb = x.shape[0]
    return pl.pallas_call(
        _tc_body,
        grid=(nb, _TCNB),
        in_specs=[pl.BlockSpec((1, _TCK, D2), lambda b, k: (b, k, 0))],
        out_specs=pl.BlockSpec((1, 1, D2), lambda b, k: (b, 0, 0)),
        out_shape=jax.ShapeDtypeStruct((nb, 1, D2), jnp.int32),
        scratch_shapes=[
            pltpu.VMEM((1, D2), jnp.float32),
            pltpu.VMEM((1, D2), jnp.int32),
        ],
    )(x)


def kernel(x):
    return _tc_argmin(x).reshape(B, D2).astype(jnp.int64)


# speedup vs baseline: 1.3021x; 1.3021x over previous
import functools

import jax
import jax.numpy as jnp
from jax import lax
from jax.experimental import pallas as pl
from jax.experimental.pallas import tpu as pltpu
from jax.experimental.pallas import tpu_sc as plsc

B, D1, D2 = 8, 8192, 576
NC, NS, L = 2, 16, 16
RSPLIT = 4
RPT = D1 // RSPLIT
NG = D2 // L
RCHUNK = 64
D2P = 640
NCHUNK = RPT // RCHUNK
U = 8

_mesh = plsc.VectorSubcoreMesh(
    core_axis_name="c", subcore_axis_name="s", num_cores=NC, num_subcores=NS
)


_scratch_types = [
    pltpu.VMEM((RCHUNK, D2), jnp.float32),
    pltpu.VMEM((RCHUNK, D2), jnp.float32),
    pltpu.VMEM((D2P,), jnp.float32),
    pltpu.VMEM((D2P,), jnp.int32),
    pltpu.VMEM((D2P,), jnp.float32),
    pltpu.VMEM((D2P,), jnp.int32),
    pltpu.VMEM((D2,), jnp.int32),
    pltpu.VMEM_SHARED((NS, 1, D2P), jnp.float32),
    pltpu.VMEM_SHARED((NS, 1, D2P), jnp.int32),
    pltpu.SemaphoreType.DMA,
    pltpu.SemaphoreType.DMA,
]


def _argmin_body(x_hbm, out_hbm, buf0, buf1, stv, sti, mgv, mgi, outv,
                 shv, shi, sem0, sem1):
    cid = lax.axis_index("c")
    sid = lax.axis_index("s")
    batch = cid * (NS // RSPLIT) + sid // RSPLIT
    rbase = (sid % RSPLIT) * RPT

    bufs = (buf0, buf1)
    sems = (sem0, sem1)

    def start_dma(i, buf, sem):
        return pltpu.async_copy(
            x_hbm.at[batch, pl.ds(rbase + i * RCHUNK, RCHUNK), :], buf, sem
        )

    def init_body(g, _):
        goff = g * L
        stv[pl.ds(goff, L)] = jnp.full((L,), jnp.inf, jnp.float32)
        sti[pl.ds(goff, L)] = jnp.zeros((L,), jnp.int32)
        return 0

    lax.fori_loop(0, NG, init_body, 0)

    def wait_dma(buf, sem):
        pltpu.make_async_copy(
            x_hbm.at[batch, pl.ds(0, RCHUNK), :], buf, sem
        ).wait()

    def process(buf, ci):
        rv = jnp.full((L,), rbase + ci * RCHUNK, jnp.int32)
        rvs = [rv + jnp.full((L,), sb * U, jnp.int32) for sb in range(RCHUNK // U)]

        def g_body(g, _):
            goff = g * L
            mv = stv[pl.ds(goff, L)]
            mi = sti[pl.ds(goff, L)]
            for sb in range(RCHUNK // U):
                v = [buf[sb * U + j, pl.ds(goff, L)] for j in range(U)]
                ix = [jnp.full((L,), j, jnp.int32) for j in range(U)]
                n = U
                while n > 1:
                    nv, ni = [], []
                    for p in range(0, n, 2):
                        m = v[p] <= v[p + 1]
                        nv.append(jnp.minimum(v[p], v[p + 1]))
                        ni.append(jnp.where(m, ix[p], ix[p + 1]))
                    v, ix, n = nv, ni, n // 2
                m = v[0] < mv
                mv = jnp.where(m, v[0], mv)
                mi = jnp.where(m, rvs[sb] + ix[0], mi)
            stv[pl.ds(goff, L)] = mv
            sti[pl.ds(goff, L)] = mi
            return 0

        lax.fori_loop(0, NG, g_body, 0)

    start_dma(0, bufs[0], sems[0])

    def outer_body(t, _):
        c0 = pl.multiple_of(t * 2, 2)
        wait_dma(bufs[0], sems[0])
        start_dma(c0 + 1, bufs[1], sems[1])
        process(bufs[0], c0)

        @pl.when(t < NCHUNK // 2 - 1)
        def _prefetch():
            start_dma(c0 + 2, bufs[0], sems[0])

        wait_dma(bufs[1], sems[1])
        process(bufs[1], c0 + 1)
        return 0

    lax.fori_loop(0, NCHUNK // 2, outer_body, 0)

    pltpu.sync_copy(stv, shv.at[sid, 0, :])
    pltpu.sync_copy(sti, shi.at[sid, 0, :])
    plsc.subcore_barrier()

    @pl.when(sid % RSPLIT == 0)
    def _merge():
        for j in range(1, RSPLIT):
            pltpu.sync_copy(shv.at[sid + j, 0, :], mgv)
            pltpu.sync_copy(shi.at[sid + j, 0, :], mgi)

            def m_body(g, _):
                goff = g * L
                mv = stv[pl.ds(goff, L)]
                mi = sti[pl.ds(goff, L)]
                ov = mgv[pl.ds(goff, L)]
                oi = mgi[pl.ds(goff, L)]
                m = ov < mv
                stv[pl.ds(goff, L)] = jnp.where(m, ov, mv)
                sti[pl.ds(goff, L)] = jnp.where(m, oi, mi)
                return 0

            lax.fori_loop(0, NG, m_body, 0)

        def o_body(g, _):
            goff = g * L
            outv[pl.ds(goff, L)] = sti[pl.ds(goff, L)]
            return 0

        lax.fori_loop(0, NG, o_body, 0)
        pltpu.sync_copy(outv, out_hbm.at[batch, 0, :])


_argmin_kernel = pl.kernel(
    _argmin_body,
    out_type=jax.ShapeDtypeStruct((B, 1, D2), jnp.int32),
    mesh=_mesh,
    scratch_types=_scratch_types,
)


_TCK = 512
_TCNB = D1 // _TCK


def _tc_body(x_ref, o_ref, mv_ref, mi_ref):
    k = pl.program_id(1)
    xb = x_ref[0]
    bmin = jnp.min(xb, axis=0, keepdims=True)
    iota = lax.broadcasted_iota(jnp.int32, (_TCK, D2), 0)
    bidx = jnp.min(jnp.where(xb == bmin, iota, D1), axis=0, keepdims=True)

    @pl.when(k == 0)
    def _init():
        mv_ref[...] = bmin
        mi_ref[...] = bidx

    @pl.when(k > 0)
    def _merge():
        cur = mv_ref[...]
        m = bmin < cur
        mv_ref[...] = jnp.where(m, bmin, cur)
        mi_ref[...] = jnp.where(m, bidx + k * _TCK, mi_ref[...])

    @pl.when(k == _TCNB - 1)
    def _out():
        o_ref[0] = mi_ref[...]


def _tc_argmin(x):
    nb = x.shape[0]
    return pl.pallas_call(
        _tc_body,
        grid=(nb, _TCNB),
        in_specs=[pl.BlockSpec((1, _TCK, D2), lambda b, k: (b, k, 0))],
        out_specs=pl.BlockSpec((1, 1, D2), lambda b, k: (b, 0, 0)),
        out_shape=jax.ShapeDtypeStruct((nb, 1, D2), jnp.int32),
        scratch_shapes=[
            pltpu.VMEM((1, D2), jnp.float32),
            pltpu.VMEM((1, D2), jnp.int32),
        ],
    )(x)


def _tc_body_big(x_ref, o_ref):
    xb = x_ref[0]
    bmin = jnp.min(xb, axis=0, keepdims=True)
    iota = lax.broadcasted_iota(jnp.int32, (D1, D2), 0)
    o_ref[0] = jnp.min(jnp.where(xb == bmin, iota, D1), axis=0, keepdims=True)


def _tc_argmin_big(x):
    nb = x.shape[0]
    return pl.pallas_call(
        _tc_body_big,
        grid=(nb,),
        in_specs=[pl.BlockSpec((1, D1, D2), lambda b: (b, 0, 0))],
        out_specs=pl.BlockSpec((1, 1, D2), lambda b: (b, 0, 0)),
        out_shape=jax.ShapeDtypeStruct((nb, 1, D2), jnp.int32),
    )(x)


def kernel(x):
    return _tc_argmin_big(x).reshape(B, D2).astype(jnp.int64)
